# initial kernel scaffold (unmeasured)
import jax
import jax.numpy as jnp
from jax import lax
from jax.experimental import pallas as pl
from jax.experimental.pallas import tpu as pltpu


def kernel(
    x,
):
    def body(*refs):
        pass

    out_shape = jax.ShapeDtypeStruct(..., jnp.float32)
    return pl.pallas_call(body, out_shape=out_shape)(...)


# baseline (device time: 1179491 ns/iter reference)
import functools

import jax
import jax.numpy as jnp
from jax import lax
from jax.experimental import pallas as pl
from jax.experimental.pallas import tpu as pltpu

N_DEV = 4
M_PER = 4096
M_GLOBAL = N_DEV * M_PER
N_COLS = 1024
COL_BLOCK = 128


def _bitonic_pass(x, iota, j, k, flip):
    is_lo = (iota & j) == 0
    dirbit = jnp.logical_xor((iota & k) != 0, flip)
    down = jnp.concatenate([x[j:], x[:j]], axis=0)
    a = jnp.where(dirbit, jnp.maximum(x, down), jnp.minimum(x, down))
    up = jnp.concatenate([x[-j:], x[:-j]], axis=0)
    b = jnp.where(dirbit, jnp.minimum(x, up), jnp.maximum(x, up))
    return jnp.where(is_lo, a, b)


def _presort_body(x_ref, o_ref):
    my = lax.axis_index("i")
    flip = (my % 2) == 1
    iota = lax.broadcasted_iota(jnp.int32, (M_PER, 1), 0)
    o_ref[...] = x_ref[...]
    k = 2
    while k <= M_PER:
        j = k // 2
        while j >= 1:
            o_ref[...] = _bitonic_pass(o_ref[...], iota, j, k, flip)
            j //= 2
        k *= 2


def _presort(x):
    grid = N_COLS // COL_BLOCK
    return pl.pallas_call(
        _presort_body,
        grid=(grid,),
        in_specs=[pl.BlockSpec((M_PER, COL_BLOCK), lambda c: (0, c))],
        out_specs=pl.BlockSpec((M_PER, COL_BLOCK), lambda c: (0, c)),
        out_shape=jax.ShapeDtypeStruct((M_PER, N_COLS), jnp.float32),
        compiler_params=pltpu.CompilerParams(
            vmem_limit_bytes=100 * 1024 * 1024
        ),
    )(x)


def _gather_body(x_ref, out_ref, send_sems, recv_sems, copy_sem):
    my = lax.axis_index("i")
    left = lax.rem(my + N_DEV - 1, N_DEV)
    right = lax.rem(my + 1, N_DEV)

    barrier_sem = pltpu.get_barrier_semaphore()
    for nbr in (left, right):
        pl.semaphore_signal(
            barrier_sem, inc=1,
            device_id=(nbr,), device_id_type=pl.DeviceIdType.MESH,
        )
    pl.semaphore_wait(barrier_sem, 2)

    local = pltpu.make_async_copy(
        x_ref, out_ref.at[pl.ds(my * M_PER, M_PER), :], copy_sem
    )
    local.start()
    local.wait()

    for h in range(N_DEV - 1):
        blk = lax.rem(my + N_DEV - h, N_DEV)
        sl = pl.ds(blk * M_PER, M_PER)
        rdma = pltpu.make_async_remote_copy(
            src_ref=out_ref.at[sl, :],
            dst_ref=out_ref.at[sl, :],
            send_sem=send_sems.at[h],
            recv_sem=recv_sems.at[h],
            device_id=(right,),
            device_id_type=pl.DeviceIdType.MESH,
        )
        rdma.start()
        rdma.wait()


def _allgather(xs):
    return pl.pallas_call(
        _gather_body,
        in_specs=[pl.BlockSpec(memory_space=pltpu.MemorySpace.HBM)],
        out_specs=pl.BlockSpec(memory_space=pltpu.MemorySpace.HBM),
        out_shape=jax.ShapeDtypeStruct((M_GLOBAL, N_COLS), jnp.float32),
        scratch_shapes=[
            pltpu.SemaphoreType.DMA((N_DEV - 1,)),
            pltpu.SemaphoreType.DMA((N_DEV - 1,)),
            pltpu.SemaphoreType.DMA,
        ],
        compiler_params=pltpu.CompilerParams(collective_id=0),
    )(xs)


def _cmpx_small(x, iota, j, desc):
    is_lo = (iota & j) == 0
    down = jnp.concatenate([x[j:], x[:j]], axis=0)
    a = jnp.maximum(x, down) if desc else jnp.minimum(x, down)
    up = jnp.concatenate([x[-j:], x[:-j]], axis=0)
    b = jnp.minimum(x, up) if desc else jnp.maximum(x, up)
    return jnp.where(is_lo, a, b)


def _block_pass(a_ref, j, k):
    for p in range(M_GLOBAL // (2 * j)):
        base = p * 2 * j
        desc = (base & k) != 0
        lo = a_ref[base : base + j, :]
        hi = a_ref[base + j : base + 2 * j, :]
        mn = jnp.minimum(lo, hi)
        mx = jnp.maximum(lo, hi)
        a_ref[base : base + j, :] = mx if desc else mn
        a_ref[base + j : base + 2 * j, :] = mn if desc else mx


def _slab_passes(a_ref, liota, k):
    for s in range(N_DEV):
        base = s * M_PER
        desc = (base & k) != 0
        j = M_PER // 2
        while j >= 1:
            a_ref[base : base + M_PER, :] = _cmpx_small(
                a_ref[base : base + M_PER, :], liota, j, desc
            )
            j //= 2


def _merge_body(g_ref, o_ref, a_ref):
    my = lax.axis_index("i")
    liota = lax.broadcasted_iota(jnp.int32, (M_PER, 1), 0)
    a_ref[...] = g_ref[...]
    _block_pass(a_ref, 4096, 8192)
    _slab_passes(a_ref, liota, 8192)
    _block_pass(a_ref, 8192, 16384)
    _block_pass(a_ref, 4096, 16384)
    _slab_passes(a_ref, liota, 16384)
    o_ref[...] = a_ref[pl.ds(my * M_PER, M_PER), :]


def _merge(g):
    grid = N_COLS // COL_BLOCK
    return pl.pallas_call(
        _merge_body,
        grid=(grid,),
        in_specs=[pl.BlockSpec((M_GLOBAL, COL_BLOCK), lambda c: (0, c))],
        out_specs=pl.BlockSpec((M_PER, COL_BLOCK), lambda c: (0, c)),
        out_shape=jax.ShapeDtypeStruct((M_PER, N_COLS), jnp.float32),
        scratch_shapes=[pltpu.VMEM((M_GLOBAL, COL_BLOCK), jnp.float32)],
        compiler_params=pltpu.CompilerParams(
            vmem_limit_bytes=100 * 1024 * 1024
        ),
    )(g)


def kernel(x):
    xs = _presort(x)
    g = _allgather(xs)
    return _merge(g)


# device time: 816652 ns/iter; 1.4443x vs baseline; 1.4443x over previous
import jax
import jax.numpy as jnp
from jax import lax
from jax.experimental import pallas as pl
from jax.experimental.pallas import tpu as pltpu

jax.config.update("jax_compilation_cache_dir", "/tmp/jax_comp_cache")
jax.config.update("jax_persistent_cache_min_entry_size_bytes", -1)
jax.config.update("jax_persistent_cache_min_compile_time_secs", 0.0)

N_DEV = 4
M_PER = 4096
M_HALF = M_PER // 2
M_GLOBAL = N_DEV * M_PER
N_COLS = 1024
COL_BLOCK = 128


def _presort_pass(x, iota, j, k, flip):
    is_lo = (iota & j) == 0
    take_max = jnp.logical_xor(
        jnp.logical_not(is_lo), jnp.logical_xor((iota & k) != 0, flip)
    )
    down = jnp.concatenate([x[j:], x[:j]], axis=0)
    up = jnp.concatenate([x[-j:], x[:-j]], axis=0)
    partner = jnp.where(is_lo, down, up)
    return jnp.where(take_max, jnp.maximum(x, partner), jnp.minimum(x, partner))


def _cmpx_small(x, iota, j, desc):
    is_lo = (iota & j) == 0
    down = jnp.concatenate([x[j:], x[:j]], axis=0)
    a = jnp.maximum(x, down) if desc else jnp.minimum(x, down)
    up = jnp.concatenate([x[-j:], x[:-j]], axis=0)
    b = jnp.minimum(x, up) if desc else jnp.maximum(x, up)
    return jnp.where(is_lo, a, b)


def _presort_body(x_ref, o_ref):
    my = lax.axis_index("i")
    flip = (my % 2) == 1
    iota = lax.broadcasted_iota(jnp.int32, (M_PER, 1), 0)
    o_ref[...] = x_ref[...]
    k = 2
    while k <= M_PER:
        j = k // 2
        while j >= 1:
            o_ref[...] = _presort_pass(o_ref[...], iota, j, k, flip)
            j //= 2
        k *= 2


def _presort(x):
    grid = N_COLS // COL_BLOCK
    return pl.pallas_call(
        _presort_body,
        grid=(grid,),
        in_specs=[pl.BlockSpec((M_PER, COL_BLOCK), lambda c: (0, c))],
        out_specs=pl.BlockSpec((M_PER, COL_BLOCK), lambda c: (0, c)),
        out_shape=jax.ShapeDtypeStruct((M_PER, N_COLS), jnp.float32),
        compiler_params=pltpu.CompilerParams(
            vmem_limit_bytes=100 * 1024 * 1024
        ),
    )(x)


def _gather_body(x_ref, out_ref, sr, rr, sl, rl, copy_sem):
    my = lax.axis_index("i")
    left = lax.rem(my + N_DEV - 1, N_DEV)
    right = lax.rem(my + 1, N_DEV)

    barrier_sem = pltpu.get_barrier_semaphore()
    for nbr in (left, right):
        pl.semaphore_signal(
            barrier_sem, inc=1,
            device_id=(nbr,), device_id_type=pl.DeviceIdType.MESH,
        )
    pl.semaphore_wait(barrier_sem, 2)

    local = pltpu.make_async_copy(
        x_ref, out_ref.at[pl.ds(my * M_PER, M_PER), :], copy_sem
    )
    local.start()
    local.wait()

    for h in range(N_DEV - 1):
        blk_r = lax.rem(my + N_DEV - h, N_DEV)
        blk_l = lax.rem(my + h, N_DEV)
        sl_r = pl.ds(blk_r * M_PER, M_HALF)
        sl_l = pl.ds(blk_l * M_PER + M_HALF, M_HALF)
        rdma_r = pltpu.make_async_remote_copy(
            src_ref=out_ref.at[sl_r, :],
            dst_ref=out_ref.at[sl_r, :],
            send_sem=sr.at[h],
            recv_sem=rr.at[h],
            device_id=(right,),
            device_id_type=pl.DeviceIdType.MESH,
        )
        rdma_l = pltpu.make_async_remote_copy(
            src_ref=out_ref.at[sl_l, :],
            dst_ref=out_ref.at[sl_l, :],
            send_sem=sl.at[h],
            recv_sem=rl.at[h],
            device_id=(left,),
            device_id_type=pl.DeviceIdType.MESH,
        )
        rdma_r.start()
        rdma_l.start()
        rdma_r.wait()
        rdma_l.wait()


def _allgather(xs):
    return pl.pallas_call(
        _gather_body,
        in_specs=[pl.BlockSpec(memory_space=pltpu.MemorySpace.HBM)],
        out_specs=pl.BlockSpec(memory_space=pltpu.MemorySpace.HBM),
        out_shape=jax.ShapeDtypeStruct((M_GLOBAL, N_COLS), jnp.float32),
        scratch_shapes=[
            pltpu.SemaphoreType.DMA((N_DEV - 1,)),
            pltpu.SemaphoreType.DMA((N_DEV - 1,)),
            pltpu.SemaphoreType.DMA((N_DEV - 1,)),
            pltpu.SemaphoreType.DMA((N_DEV - 1,)),
            pltpu.SemaphoreType.DMA,
        ],
        compiler_params=pltpu.CompilerParams(collective_id=0),
    )(xs)


def _block_pass(a_ref, j, k):
    for p in range(M_GLOBAL // (2 * j)):
        base = p * 2 * j
        desc = (base & k) != 0
        lo = a_ref[base : base + j, :]
        hi = a_ref[base + j : base + 2 * j, :]
        mn = jnp.minimum(lo, hi)
        mx = jnp.maximum(lo, hi)
        a_ref[base : base + j, :] = mx if desc else mn
        a_ref[base + j : base + 2 * j, :] = mn if desc else mx


def _slab_passes(a_ref, liota, k):
    for s in range(N_DEV):
        base = s * M_PER
        desc = (base & k) != 0
        j = M_PER // 2
        while j >= 1:
            a_ref[base : base + M_PER, :] = _cmpx_small(
                a_ref[base : base + M_PER, :], liota, j, desc
            )
            j //= 2


def _merge_body(g_ref, o_ref, a_ref):
    my = lax.axis_index("i")
    half = my // 2
    liota = lax.broadcasted_iota(jnp.int32, (M_PER, 1), 0)
    a_ref[...] = g_ref[...]

    _block_pass(a_ref, 4096, 8192)
    _slab_passes(a_ref, liota, 8192)

    lo = a_ref[0 : 2 * M_PER, :]
    hi = a_ref[2 * M_PER : 4 * M_PER, :]
    a_ref[pl.ds(half * 2 * M_PER, 2 * M_PER), :] = jnp.where(
        half == 0, jnp.minimum(lo, hi), jnp.maximum(lo, hi)
    )
    base = half * 2 * M_PER
    qlo = a_ref[pl.ds(base, M_PER), :]
    qhi = a_ref[pl.ds(base + M_PER, M_PER), :]
    a_ref[pl.ds(base, M_PER), :] = jnp.minimum(qlo, qhi)
    a_ref[pl.ds(base + M_PER, M_PER), :] = jnp.maximum(qlo, qhi)
    o_ref[...] = a_ref[pl.ds(my * M_PER, M_PER), :]
    j = M_PER // 2
    while j >= 1:
        o_ref[...] = _cmpx_small(o_ref[...], liota, j, False)
        j //= 2


def _merge(g):
    grid = N_COLS // COL_BLOCK
    return pl.pallas_call(
        _merge_body,
        grid=(grid,),
        in_specs=[pl.BlockSpec((M_GLOBAL, COL_BLOCK), lambda c: (0, c))],
        out_specs=pl.BlockSpec((M_PER, COL_BLOCK), lambda c: (0, c)),
        out_shape=jax.ShapeDtypeStruct((M_PER, N_COLS), jnp.float32),
        scratch_shapes=[pltpu.VMEM((M_GLOBAL, COL_BLOCK), jnp.float32)],
        compiler_params=pltpu.CompilerParams(
            vmem_limit_bytes=100 * 1024 * 1024
        ),
    )(g)


def kernel(x):
    xs = _presort(x)
    g = _allgather(xs)
    return _merge(g)


# device time: 499421 ns/iter; 2.3617x vs baseline; 1.6352x over previous
import jax
import jax.numpy as jnp
from jax import lax
from jax.experimental import pallas as pl
from jax.experimental.pallas import tpu as pltpu

jax.config.update("jax_compilation_cache_dir", "/tmp/jax_comp_cache")
jax.config.update("jax_persistent_cache_min_entry_size_bytes", -1)
jax.config.update("jax_persistent_cache_min_compile_time_secs", 0.0)

N_DEV = 4
M_PER = 4096
M_HALF = M_PER // 2
M_GLOBAL = N_DEV * M_PER
N_COLS = 1024
COL_BLOCK = 128
N_TILES = N_COLS // COL_BLOCK
N_HOPS = N_DEV - 1


def _presort_pass(x, iota, j, k, flip):
    is_lo = (iota & j) == 0
    take_max = jnp.logical_xor(
        jnp.logical_not(is_lo), jnp.logical_xor((iota & k) != 0, flip)
    )
    down = jnp.concatenate([x[j:], x[:j]], axis=0)
    up = jnp.concatenate([x[-j:], x[:-j]], axis=0)
    partner = jnp.where(is_lo, down, up)
    return jnp.where(take_max, jnp.maximum(x, partner), jnp.minimum(x, partner))


def _cmpx_small(x, iota, j, desc):
    is_lo = (iota & j) == 0
    down = jnp.concatenate([x[j:], x[:j]], axis=0)
    a = jnp.maximum(x, down) if desc else jnp.minimum(x, down)
    up = jnp.concatenate([x[-j:], x[:-j]], axis=0)
    b = jnp.minimum(x, up) if desc else jnp.maximum(x, up)
    return jnp.where(is_lo, a, b)


def _pg_body(x_ref, g_ref, bufs, in_sem, oc, sr, rr, slm, rl):
    my = lax.axis_index("i")
    left = lax.rem(my + N_DEV - 1, N_DEV)
    right = lax.rem(my + 1, N_DEV)
    flip = (my % 2) == 1
    iota = lax.broadcasted_iota(jnp.int32, (M_PER, 1), 0)

    barrier_sem = pltpu.get_barrier_semaphore()
    for nbr in (left, right):
        pl.semaphore_signal(
            barrier_sem, inc=1,
            device_id=(nbr,), device_id_type=pl.DeviceIdType.MESH,
        )
    pl.semaphore_wait(barrier_sem, 2)

    rdmas = {}

    def start_hop(h, t):
        blk_r = lax.rem(my + N_DEV - h, N_DEV)
        blk_l = lax.rem(my + h, N_DEV)
        sl_r = pl.ds(blk_r * M_PER, M_HALF)
        sl_l = pl.ds(blk_l * M_PER + M_HALF, M_HALF)
        rdma_r = pltpu.make_async_remote_copy(
            src_ref=g_ref.at[t, sl_r, :],
            dst_ref=g_ref.at[t, sl_r, :],
            send_sem=sr.at[h, t],
            recv_sem=rr.at[h, t],
            device_id=(right,),
            device_id_type=pl.DeviceIdType.MESH,
        )
        rdma_l = pltpu.make_async_remote_copy(
            src_ref=g_ref.at[t, sl_l, :],
            dst_ref=g_ref.at[t, sl_l, :],
            send_sem=slm.at[h, t],
            recv_sem=rl.at[h, t],
            device_id=(left,),
            device_id_type=pl.DeviceIdType.MESH,
        )
        rdma_r.start()
        rdma_l.start()
        rdmas[(h, t)] = (rdma_r, rdma_l)

    def wait_recv(h, t):
        rdma_r, rdma_l = rdmas[(h, t)]
        rdma_r.wait_recv()
        rdma_l.wait_recv()

    for t in range(N_TILES):
        buf = bufs.at[t % 2]
        ld = pltpu.make_async_copy(
            x_ref.at[:, pl.ds(t * COL_BLOCK, COL_BLOCK)], buf, in_sem
        )
        ld.start()
        ld.wait()
        k = 2
        while k <= M_PER:
            j = k // 2
            while j >= 1:
                buf[...] = _presort_pass(buf[...], iota, j, k, flip)
                j //= 2
            k *= 2
        st = pltpu.make_async_copy(
            buf, g_ref.at[t, pl.ds(my * M_PER, M_PER), :], oc.at[t % 2]
        )
        st.start()
        st.wait()
        start_hop(0, t)
        if t >= 1:
            wait_recv(0, t - 1)
            start_hop(1, t - 1)
        if t >= 2:
            wait_recv(1, t - 2)
            start_hop(2, t - 2)
        if t >= 3:
            wait_recv(2, t - 3)

    T = N_TILES
    wait_recv(0, T - 1)
    start_hop(1, T - 1)
    wait_recv(1, T - 2)
    start_hop(2, T - 2)
    wait_recv(1, T - 1)
    start_hop(2, T - 1)
    wait_recv(2, T - 3)
    wait_recv(2, T - 2)
    wait_recv(2, T - 1)

    for pair in rdmas.values():
        pair[0].wait_send()
        pair[1].wait_send()


def _presort_gather(x):
    return pl.pallas_call(
        _pg_body,
        in_specs=[pl.BlockSpec(memory_space=pltpu.MemorySpace.HBM)],
        out_specs=pl.BlockSpec(memory_space=pltpu.MemorySpace.HBM),
        out_shape=jax.ShapeDtypeStruct(
            (N_TILES, M_GLOBAL, COL_BLOCK), jnp.float32
        ),
        scratch_shapes=[
            pltpu.VMEM((2, M_PER, COL_BLOCK), jnp.float32),
            pltpu.SemaphoreType.DMA,
            pltpu.SemaphoreType.DMA((2,)),
            pltpu.SemaphoreType.DMA((N_HOPS, N_TILES)),
            pltpu.SemaphoreType.DMA((N_HOPS, N_TILES)),
            pltpu.SemaphoreType.DMA((N_HOPS, N_TILES)),
            pltpu.SemaphoreType.DMA((N_HOPS, N_TILES)),
        ],
        compiler_params=pltpu.CompilerParams(
            collective_id=0,
            vmem_limit_bytes=100 * 1024 * 1024,
        ),
    )(x)


def _merge_body(g_ref, o_ref, a_ref):
    my = lax.axis_index("i")
    half = my // 2
    liota = lax.broadcasted_iota(jnp.int32, (M_PER, 1), 0)
    a_ref[...] = g_ref[0]

    for p in range(2):
        base = p * 8192
        desc = p == 1
        lo = a_ref[base : base + M_PER, :]
        hi = a_ref[base + M_PER : base + 2 * M_PER, :]
        mn = jnp.minimum(lo, hi)
        mx = jnp.maximum(lo, hi)
        a_ref[base : base + M_PER, :] = mx if desc else mn
        a_ref[base + M_PER : base + 2 * M_PER, :] = mn if desc else mx
    for s in range(N_DEV):
        base = s * M_PER
        desc = (base & 8192) != 0
        j = M_PER // 2
        while j >= 1:
            a_ref[base : base + M_PER, :] = _cmpx_small(
                a_ref[base : base + M_PER, :], liota, j, desc
            )
            j //= 2

    lo = a_ref[0 : 2 * M_PER, :]
    hi = a_ref[2 * M_PER : 4 * M_PER, :]
    a_ref[pl.ds(half * 2 * M_PER, 2 * M_PER), :] = jnp.where(
        half == 0, jnp.minimum(lo, hi), jnp.maximum(lo, hi)
    )
    base = half * 2 * M_PER
    qlo = a_ref[pl.ds(base, M_PER), :]
    qhi = a_ref[pl.ds(base + M_PER, M_PER), :]
    a_ref[pl.ds(base, M_PER), :] = jnp.minimum(qlo, qhi)
    a_ref[pl.ds(base + M_PER, M_PER), :] = jnp.maximum(qlo, qhi)
    o_ref[...] = a_ref[pl.ds(my * M_PER, M_PER), :]
    j = M_PER // 2
    while j >= 1:
        o_ref[...] = _cmpx_small(o_ref[...], liota, j, False)
        j //= 2


def _merge(g):
    return pl.pallas_call(
        _merge_body,
        grid=(N_TILES,),
        in_specs=[
            pl.BlockSpec((1, M_GLOBAL, COL_BLOCK), lambda c: (c, 0, 0))
        ],
        out_specs=pl.BlockSpec((M_PER, COL_BLOCK), lambda c: (0, c)),
        out_shape=jax.ShapeDtypeStruct((M_PER, N_COLS), jnp.float32),
        scratch_shapes=[pltpu.VMEM((M_GLOBAL, COL_BLOCK), jnp.float32)],
        compiler_params=pltpu.CompilerParams(
            vmem_limit_bytes=100 * 1024 * 1024
        ),
    )(g)


def kernel(x):
    g = _presort_gather(x)
    return _merge(g)
